# baseline (speedup 1.0000x reference)
import jax
import jax.numpy as jnp
from jax import lax
from jax.experimental import pallas as pl
from jax.experimental.pallas import tpu as pltpu
from jax.experimental.pallas import tpu_sc as plsc

NC = 2
NS = 16
CK = 128
NBUF = 2


def _sc_segment_sum(n_pad, rows_per_sub, ch):
  mesh = plsc.VectorSubcoreMesh(core_axis_name="c", subcore_axis_name="s")

  def body(xlo_hbm, xhi_hbm, src_hbm, dst_hbm, outlo_hbm, outhi_hbm,
           zbuf, srcv, dstv, r0, r1, acc, sg0, sg1):
    rows = (r0, r1)
    sg = (sg0, sg1)
    cid = lax.axis_index("c")
    sid = lax.axis_index("s")

    zero16 = jnp.zeros((16,), jnp.float32)
    zrows = rows_per_sub // 8

    def zbody(i, _):
      for j in range(4):
        zbuf[i, pl.ds(j * 16, 16)] = zero16
      return 0

    pltpu.async_copy(src_hbm.at[sid], srcv, sg0)
    pltpu.async_copy(dst_hbm.at[sid], dstv, sg1)
    lax.fori_loop(0, zrows, zbody, 0)
    for j in range(8):
      pltpu.sync_copy(zbuf, acc.at[pl.ds(sid * rows_per_sub + j * zrows, zrows)])
    pltpu.make_async_copy(src_hbm.at[sid], srcv, sg0).wait()
    pltpu.make_async_copy(dst_hbm.at[sid], dstv, sg1).wait()
    plsc.subcore_barrier()

    def run(x_ref):
      for b in range(NBUF):
        pltpu.async_copy(x_ref.at[srcv.at[b]], rows[b], sg[b])

      def gbody(g, _):
        c0 = g * NBUF
        for b in range(NBUF):
          c = c0 + b
          pltpu.make_async_copy(x_ref.at[srcv.at[c]], rows[b], sg[b]).wait()
          pltpu.sync_copy(rows[b], acc.at[dstv.at[c]], add=True)
          pltpu.async_copy(x_ref.at[srcv.at[c + NBUF]], rows[b], sg[b])
        return 0

      lax.fori_loop(0, ch // NBUF - 1, gbody, 0)
      c0 = ch - NBUF
      for b in range(NBUF):
        c = c0 + b
        pltpu.make_async_copy(x_ref.at[srcv.at[c]], rows[b], sg[b]).wait()
        pltpu.sync_copy(rows[b], acc.at[dstv.at[c]], add=True)

    pl.when(cid == 0)(lambda: run(xlo_hbm))
    pl.when(cid == 1)(lambda: run(xhi_hbm))
    plsc.subcore_barrier()

    sl = pl.ds(sid * rows_per_sub, rows_per_sub)
    pl.when(cid == 0)(lambda: pltpu.sync_copy(acc.at[sl], outlo_hbm.at[sl]))
    pl.when(cid == 1)(lambda: pltpu.sync_copy(acc.at[sl], outhi_hbm.at[sl]))

  return pl.kernel(
      body,
      out_type=(
          jax.ShapeDtypeStruct((n_pad, 64), jnp.float32),
          jax.ShapeDtypeStruct((n_pad, 64), jnp.float32),
      ),
      mesh=mesh,
      compiler_params=pltpu.CompilerParams(use_tc_tiling_on_sc=False),
      scratch_types=[
          pltpu.VMEM((rows_per_sub // 8, 64), jnp.float32),
          pltpu.VMEM((ch, CK), jnp.int32),
          pltpu.VMEM((ch, CK), jnp.int32),
          pltpu.VMEM((CK, 64), jnp.float32),
          pltpu.VMEM((CK, 64), jnp.float32),
          pltpu.VMEM_SHARED((n_pad, 64), jnp.float32),
          pltpu.SemaphoreType.DMA,
          pltpu.SemaphoreType.DMA,
      ],
  )


def _tc_body(plo_ref, phi_ref, w_ref, b_ref, o_ref):
  o_ref[...] = (
      jnp.dot(plo_ref[...], w_ref[0:64, :], preferred_element_type=jnp.float32)
      + jnp.dot(phi_ref[...], w_ref[64:128, :], preferred_element_type=jnp.float32)
      + b_ref[...]
  )


def _tc_combine_matmul(plo, phi, weight, bias, n):
  br = 1000
  return pl.pallas_call(
      _tc_body,
      grid=(n // br,),
      in_specs=[
          pl.BlockSpec((br, 64), lambda i: (i, 0)),
          pl.BlockSpec((br, 64), lambda i: (i, 0)),
          pl.BlockSpec((128, 128), lambda i: (0, 0)),
          pl.BlockSpec((1, 128), lambda i: (0, 0)),
      ],
      out_specs=pl.BlockSpec((br, 128), lambda i: (i, 0)),
      out_shape=jax.ShapeDtypeStruct((n, 128), jnp.float32),
  )(plo, phi, weight, bias.reshape(1, 128))


@jax.jit
def kernel(x, edge_index, weight, bias):
  n, d = x.shape
  e = edge_index.shape[1]
  assert d == 128 and weight.shape == (128, 128)

  ch = NBUF * (-(-e // (NS * CK * NBUF)))
  e_pad = NS * ch * CK
  rows_per_sub = 8 * (-(-(n + 1) // (NS * 8)))
  n_pad = rows_per_sub * NS

  src = edge_index[0]
  dst = edge_index[1]
  pad = e_pad - e
  src_p = jnp.concatenate([src, jnp.zeros((pad,), jnp.int32)]).reshape(NS, ch, CK)
  pad_dst = n + jnp.arange(pad, dtype=jnp.int32) % (n_pad - n)
  dst_p = jnp.concatenate([dst, pad_dst]).reshape(NS, ch, CK)
  x_lo = x[:, :64]
  x_hi = x[:, 64:]

  plo, phi = _sc_segment_sum(n_pad, rows_per_sub, ch)(x_lo, x_hi, src_p, dst_p)
  return _tc_combine_matmul(plo, phi, weight, bias, n)


# speedup vs baseline: 1.0495x; 1.0495x over previous
import jax
import jax.numpy as jnp
from jax import lax
from jax.experimental import pallas as pl
from jax.experimental.pallas import tpu as pltpu
from jax.experimental.pallas import tpu_sc as plsc

NC = 2
NS = 16
CK = 96
NBUF = 2


def _sc_segment_sum(n_pad, rows_per_sub, ch):
  mesh = plsc.VectorSubcoreMesh(core_axis_name="c", subcore_axis_name="s")

  def body(xlo_hbm, xhi_hbm, src_hbm, dst_hbm, outlo_hbm, outhi_hbm,
           zbuf, srcv, dstv, r0, r1, acc, sg0, sg1):
    rows = (r0, r1)
    sg = (sg0, sg1)
    cid = lax.axis_index("c")
    sid = lax.axis_index("s")

    zero16 = jnp.zeros((16,), jnp.float32)
    zrows = rows_per_sub // 8

    def zbody(i, _):
      for j in range(4):
        zbuf[i, pl.ds(j * 16, 16)] = zero16
      return 0

    pltpu.async_copy(src_hbm.at[sid], srcv, sg0)
    pltpu.async_copy(dst_hbm.at[sid], dstv, sg1)
    lax.fori_loop(0, zrows, zbody, 0)
    for j in range(8):
      pltpu.sync_copy(zbuf, acc.at[pl.ds(sid * rows_per_sub + j * zrows, zrows)])
    pltpu.make_async_copy(src_hbm.at[sid], srcv, sg0).wait()
    pltpu.make_async_copy(dst_hbm.at[sid], dstv, sg1).wait()
    plsc.subcore_barrier()

    def run(x_ref):
      for b in range(NBUF):
        pltpu.async_copy(x_ref.at[srcv.at[b]], rows[b], sg[b])

      def gbody(g, _):
        c0 = g * NBUF
        for b in range(NBUF):
          c = c0 + b
          pltpu.make_async_copy(x_ref.at[srcv.at[c]], rows[b], sg[b]).wait()
          pltpu.sync_copy(rows[b], acc.at[dstv.at[c]], add=True)
          pltpu.async_copy(x_ref.at[srcv.at[c + NBUF]], rows[b], sg[b])
        return 0

      lax.fori_loop(0, ch // NBUF - 1, gbody, 0)
      c0 = ch - NBUF
      for b in range(NBUF):
        c = c0 + b
        pltpu.make_async_copy(x_ref.at[srcv.at[c]], rows[b], sg[b]).wait()
        pltpu.sync_copy(rows[b], acc.at[dstv.at[c]], add=True)

    pl.when(cid == 0)(lambda: run(xlo_hbm))
    pl.when(cid == 1)(lambda: run(xhi_hbm))
    plsc.subcore_barrier()

    sl = pl.ds(sid * rows_per_sub, rows_per_sub)
    pl.when(cid == 0)(lambda: pltpu.sync_copy(acc.at[sl], outlo_hbm.at[sl]))
    pl.when(cid == 1)(lambda: pltpu.sync_copy(acc.at[sl], outhi_hbm.at[sl]))

  return pl.kernel(
      body,
      out_type=(
          jax.ShapeDtypeStruct((n_pad, 64), jnp.float32),
          jax.ShapeDtypeStruct((n_pad, 64), jnp.float32),
      ),
      mesh=mesh,
      compiler_params=pltpu.CompilerParams(use_tc_tiling_on_sc=False),
      scratch_types=[
          pltpu.VMEM((rows_per_sub // 8, 64), jnp.float32),
          pltpu.VMEM((ch, CK), jnp.int32),
          pltpu.VMEM((ch, CK), jnp.int32),
          pltpu.VMEM((CK, 64), jnp.float32),
          pltpu.VMEM((CK, 64), jnp.float32),
          pltpu.VMEM_SHARED((n_pad, 64), jnp.float32),
          pltpu.SemaphoreType.DMA,
          pltpu.SemaphoreType.DMA,
      ],
  )


def _tc_body(plo_ref, phi_ref, w_ref, b_ref, o_ref):
  o_ref[...] = (
      jnp.dot(plo_ref[...], w_ref[0:64, :], preferred_element_type=jnp.float32)
      + jnp.dot(phi_ref[...], w_ref[64:128, :], preferred_element_type=jnp.float32)
      + b_ref[...]
  )


def _tc_combine_matmul(plo, phi, weight, bias, n):
  br = 1000
  return pl.pallas_call(
      _tc_body,
      grid=(n // br,),
      in_specs=[
          pl.BlockSpec((br, 64), lambda i: (i, 0)),
          pl.BlockSpec((br, 64), lambda i: (i, 0)),
          pl.BlockSpec((128, 128), lambda i: (0, 0)),
          pl.BlockSpec((1, 128), lambda i: (0, 0)),
      ],
      out_specs=pl.BlockSpec((br, 128), lambda i: (i, 0)),
      out_shape=jax.ShapeDtypeStruct((n, 128), jnp.float32),
  )(plo, phi, weight, bias.reshape(1, 128))


@jax.jit
def kernel(x, edge_index, weight, bias):
  n, d = x.shape
  e = edge_index.shape[1]
  assert d == 128 and weight.shape == (128, 128)

  ch = NBUF * (-(-e // (NS * CK * NBUF)))
  e_pad = NS * ch * CK
  rows_per_sub = 8 * (-(-(n + 1) // (NS * 8)))
  n_pad = rows_per_sub * NS

  src = edge_index[0]
  dst = edge_index[1]
  pad = e_pad - e
  src_p = jnp.concatenate([src, jnp.zeros((pad,), jnp.int32)]).reshape(NS, ch, CK)
  pad_dst = n + jnp.arange(pad, dtype=jnp.int32) % (n_pad - n)
  dst_p = jnp.concatenate([dst, pad_dst]).reshape(NS, ch, CK)
  x_lo = x[:, :64]
  x_hi = x[:, 64:]

  plo, phi = _sc_segment_sum(n_pad, rows_per_sub, ch)(x_lo, x_hi, src_p, dst_p)
  return _tc_combine_matmul(plo, phi, weight, bias, n)


# speedup vs baseline: 1.2474x; 1.1886x over previous
import jax
import jax.numpy as jnp
from jax import lax
from jax.experimental import pallas as pl
from jax.experimental.pallas import tpu as pltpu
from jax.experimental.pallas import tpu_sc as plsc

NC = 2
NS = 16
CK = 80
NBUF = 2


def _sc_segment_sum(n_pad, rows_per_sub, ch):
  mesh = plsc.VectorSubcoreMesh(core_axis_name="c", subcore_axis_name="s")

  def body(xlo_hbm, xhi_hbm, src_hbm, dst_hbm, outlo_hbm, outhi_hbm,
           zbuf, srcv, dstv, r0, r1, acc, sg0, sg1):
    rows = (r0, r1)
    sg = (sg0, sg1)
    cid = lax.axis_index("c")
    sid = lax.axis_index("s")

    zero16 = jnp.zeros((16,), jnp.float32)
    zrows = rows_per_sub // 8

    def zbody(i, _):
      for j in range(4):
        zbuf[i, pl.ds(j * 16, 16)] = zero16
      return 0

    pltpu.async_copy(src_hbm.at[sid], srcv, sg0)
    pltpu.async_copy(dst_hbm.at[sid], dstv, sg1)
    lax.fori_loop(0, zrows, zbody, 0)
    for j in range(8):
      pltpu.sync_copy(zbuf, acc.at[pl.ds(sid * rows_per_sub + j * zrows, zrows)])
    pltpu.make_async_copy(src_hbm.at[sid], srcv, sg0).wait()
    pltpu.make_async_copy(dst_hbm.at[sid], dstv, sg1).wait()
    plsc.subcore_barrier()

    def run(x_ref):
      for b in range(NBUF):
        pltpu.async_copy(x_ref.at[srcv.at[b]], rows[b], sg[b])

      def gbody(g, _):
        c0 = g * NBUF
        for b in range(NBUF):
          c = c0 + b
          pltpu.make_async_copy(x_ref.at[srcv.at[c]], rows[b], sg[b]).wait()
          pltpu.sync_copy(rows[b], acc.at[dstv.at[c]], add=True)
          pltpu.async_copy(x_ref.at[srcv.at[c + NBUF]], rows[b], sg[b])
        return 0

      lax.fori_loop(0, ch // NBUF - 1, gbody, 0)
      c0 = ch - NBUF
      for b in range(NBUF):
        c = c0 + b
        pltpu.make_async_copy(x_ref.at[srcv.at[c]], rows[b], sg[b]).wait()
        pltpu.sync_copy(rows[b], acc.at[dstv.at[c]], add=True)

    pl.when(cid == 0)(lambda: run(xlo_hbm))
    pl.when(cid == 1)(lambda: run(xhi_hbm))
    plsc.subcore_barrier()

    sl = pl.ds(sid * rows_per_sub, rows_per_sub)
    pl.when(cid == 0)(lambda: pltpu.sync_copy(acc.at[sl], outlo_hbm.at[sl]))
    pl.when(cid == 1)(lambda: pltpu.sync_copy(acc.at[sl], outhi_hbm.at[sl]))

  return pl.kernel(
      body,
      out_type=(
          jax.ShapeDtypeStruct((n_pad, 64), jnp.float32),
          jax.ShapeDtypeStruct((n_pad, 64), jnp.float32),
      ),
      mesh=mesh,
      compiler_params=pltpu.CompilerParams(use_tc_tiling_on_sc=False),
      scratch_types=[
          pltpu.VMEM((rows_per_sub // 8, 64), jnp.float32),
          pltpu.VMEM((ch, CK), jnp.int32),
          pltpu.VMEM((ch, CK), jnp.int32),
          pltpu.VMEM((CK, 64), jnp.float32),
          pltpu.VMEM((CK, 64), jnp.float32),
          pltpu.VMEM_SHARED((n_pad, 64), jnp.float32),
          pltpu.SemaphoreType.DMA,
          pltpu.SemaphoreType.DMA,
      ],
  )


def _tc_body(plo_ref, phi_ref, w_ref, b_ref, o_ref):
  o_ref[...] = (
      jnp.dot(plo_ref[...], w_ref[0:64, :], preferred_element_type=jnp.float32)
      + jnp.dot(phi_ref[...], w_ref[64:128, :], preferred_element_type=jnp.float32)
      + b_ref[...]
  )


def _tc_combine_matmul(plo, phi, weight, bias, n):
  br = 1000
  return pl.pallas_call(
      _tc_body,
      grid=(n // br,),
      in_specs=[
          pl.BlockSpec((br, 64), lambda i: (i, 0)),
          pl.BlockSpec((br, 64), lambda i: (i, 0)),
          pl.BlockSpec((128, 128), lambda i: (0, 0)),
          pl.BlockSpec((1, 128), lambda i: (0, 0)),
      ],
      out_specs=pl.BlockSpec((br, 128), lambda i: (i, 0)),
      out_shape=jax.ShapeDtypeStruct((n, 128), jnp.float32),
  )(plo, phi, weight, bias.reshape(1, 128))


@jax.jit
def kernel(x, edge_index, weight, bias):
  n, d = x.shape
  e = edge_index.shape[1]
  assert d == 128 and weight.shape == (128, 128)

  ch = NBUF * (-(-e // (NS * CK * NBUF)))
  e_pad = NS * ch * CK
  rows_per_sub = 8 * (-(-(n + 1) // (NS * 8)))
  n_pad = rows_per_sub * NS

  src = edge_index[0]
  dst = edge_index[1]
  pad = e_pad - e
  src_p = jnp.concatenate([src, jnp.zeros((pad,), jnp.int32)]).reshape(NS, ch, CK)
  pad_dst = n + jnp.arange(pad, dtype=jnp.int32) % (n_pad - n)
  dst_p = jnp.concatenate([dst, pad_dst]).reshape(NS, ch, CK)
  x_lo = x[:, :64]
  x_hi = x[:, 64:]

  plo, phi = _sc_segment_sum(n_pad, rows_per_sub, ch)(x_lo, x_hi, src_p, dst_p)
  return _tc_combine_matmul(plo, phi, weight, bias, n)


# speedup vs baseline: 1.2488x; 1.0011x over previous
import jax
import jax.numpy as jnp
from jax import lax
from jax.experimental import pallas as pl
from jax.experimental.pallas import tpu as pltpu
from jax.experimental.pallas import tpu_sc as plsc

NC = 2
NS = 16
CK = 80
NBUF = 2


def _sc_segment_sum(n_pad, rows_per_sub, ch):
  mesh = plsc.VectorSubcoreMesh(core_axis_name="c", subcore_axis_name="s")

  def body(xlo_hbm, xhi_hbm, src_hbm, dst_hbm, outlo_hbm, outhi_hbm,
           zbuf, srcv, dstv, r0, r1, acc, sg0, sg1):
    rows = (r0, r1)
    sg = (sg0, sg1)
    cid = lax.axis_index("c")
    sid = lax.axis_index("s")

    zero16 = jnp.zeros((16,), jnp.float32)
    zrows = rows_per_sub // 8

    def zbody(i, _):
      for j in range(4):
        zbuf[i, pl.ds(j * 16, 16)] = zero16
      return 0

    pltpu.async_copy(src_hbm.at[sid], srcv, sg0)
    pltpu.async_copy(dst_hbm.at[sid], dstv, sg1)
    lax.fori_loop(0, zrows, zbody, 0)
    for j in range(8):
      pltpu.sync_copy(zbuf, acc.at[pl.ds(sid * rows_per_sub + j * zrows, zrows)])
    pltpu.make_async_copy(src_hbm.at[sid], srcv, sg0).wait()
    pltpu.make_async_copy(dst_hbm.at[sid], dstv, sg1).wait()
    plsc.subcore_barrier()

    def run(x_ref):
      for b in range(NBUF):
        pltpu.async_copy(x_ref.at[srcv.at[b]], rows[b], sg[b])

      def gbody(g, _):
        c0 = g * NBUF
        for b in range(NBUF):
          c = c0 + b
          pltpu.make_async_copy(x_ref.at[srcv.at[c]], rows[b], sg[b]).wait()
          pltpu.sync_copy(rows[b], acc.at[dstv.at[c]], add=True)
          pltpu.async_copy(x_ref.at[srcv.at[c + NBUF]], rows[b], sg[b])
        return 0

      lax.fori_loop(0, ch // NBUF - 1, gbody, 0)
      c0 = ch - NBUF
      for b in range(NBUF):
        c = c0 + b
        pltpu.make_async_copy(x_ref.at[srcv.at[c]], rows[b], sg[b]).wait()
        pltpu.sync_copy(rows[b], acc.at[dstv.at[c]], add=True)

    pl.when(cid == 0)(lambda: run(xlo_hbm))
    pl.when(cid == 1)(lambda: run(xhi_hbm))
    plsc.subcore_barrier()

    sl = pl.ds(sid * rows_per_sub, rows_per_sub)
    pl.when(cid == 0)(lambda: pltpu.sync_copy(acc.at[sl], outlo_hbm.at[sl]))
    pl.when(cid == 1)(lambda: pltpu.sync_copy(acc.at[sl], outhi_hbm.at[sl]))

  return pl.kernel(
      body,
      out_type=(
          jax.ShapeDtypeStruct((n_pad, 64), jnp.float32),
          jax.ShapeDtypeStruct((n_pad, 64), jnp.float32),
      ),
      mesh=mesh,
      compiler_params=pltpu.CompilerParams(use_tc_tiling_on_sc=False),
      scratch_types=[
          pltpu.VMEM((rows_per_sub // 8, 64), jnp.float32),
          pltpu.VMEM((ch, CK), jnp.int32),
          pltpu.VMEM((ch, CK), jnp.int32),
          pltpu.VMEM((CK, 64), jnp.float32),
          pltpu.VMEM((CK, 64), jnp.float32),
          pltpu.VMEM_SHARED((n_pad, 64), jnp.float32),
          pltpu.SemaphoreType.DMA,
          pltpu.SemaphoreType.DMA,
      ],
  )


def _tc_body(plo_ref, phi_ref, w_ref, b_ref, o_ref):
  o_ref[...] = (
      jnp.dot(plo_ref[...], w_ref[0:64, :], preferred_element_type=jnp.float32)
      + jnp.dot(phi_ref[...], w_ref[64:128, :], preferred_element_type=jnp.float32)
      + b_ref[...]
  )


def _tc_combine_matmul(plo, phi, weight, bias, n):
  br = 1000
  return pl.pallas_call(
      _tc_body,
      grid=(n // br,),
      in_specs=[
          pl.BlockSpec((br, 64), lambda i: (i, 0)),
          pl.BlockSpec((br, 64), lambda i: (i, 0)),
          pl.BlockSpec((128, 128), lambda i: (0, 0)),
          pl.BlockSpec((1, 128), lambda i: (0, 0)),
      ],
      out_specs=pl.BlockSpec((br, 128), lambda i: (i, 0)),
      out_shape=jax.ShapeDtypeStruct((n, 128), jnp.float32),
  )(plo, phi, weight, bias.reshape(1, 128))


@jax.jit
def kernel(x, edge_index, weight, bias):
  n, d = x.shape
  e = edge_index.shape[1]
  assert d == 128 and weight.shape == (128, 128)

  ch = NBUF * (-(-e // (NS * CK * NBUF)))
  e_pad = NS * ch * CK
  rows_per_sub = 8 * (-(-(n + 1) // (NS * 8)))
  n_pad = rows_per_sub * NS

  src = edge_index[0]
  dst = edge_index[1]
  pad = e_pad - e
  if pad:
    src = jnp.concatenate([src, jnp.zeros((pad,), jnp.int32)])
    dst = jnp.concatenate(
        [dst, n + jnp.arange(pad, dtype=jnp.int32) % (n_pad - n)])
  src_p = src.reshape(NS, ch, CK)
  dst_p = dst.reshape(NS, ch, CK)
  x_lo = x[:, :64]
  x_hi = x[:, 64:]

  plo, phi = _sc_segment_sum(n_pad, rows_per_sub, ch)(x_lo, x_hi, src_p, dst_p)
  return _tc_combine_matmul(plo, phi, weight, bias, n)
